# baseline (speedup 1.0000x reference)
import jax
import jax.numpy as jnp
from jax.experimental import pallas as pl

_C = 21
_B = 32
_P = 8732
_O = 10
_THR = 0.5
_NEGPOS = 3.0
_V0 = 0.1
_V1 = 0.2


def _phase1(tgt_ref, tgtT_hi_ref, tgtT_lo_ref, pri_ref, loc_ref, conf_ref,
            v_ref, part_ref):
    tgt = tgt_ref[0]
    tgtT_hi = tgtT_hi_ref[0]
    tgtT_lo = tgtT_lo_ref[0]
    tx1 = tgt[:, 0:1]
    ty1 = tgt[:, 1:2]
    tx2 = tgt[:, 2:3]
    ty2 = tgt[:, 3:4]

    pri = pri_ref[...]
    pcx = pri[0:1, :]
    pcy = pri[1:2, :]
    pw = pri[2:3, :]
    ph = pri[3:4, :]
    px1 = pcx - pw * 0.5
    py1 = pcy - ph * 0.5
    px2 = pcx + pw * 0.5
    py2 = pcy + ph * 0.5

    iw = jnp.maximum(jnp.minimum(tx2, px2) - jnp.maximum(tx1, px1), 0.0)
    ih = jnp.maximum(jnp.minimum(ty2, py2) - jnp.maximum(ty1, py1), 0.0)
    inter = iw * ih
    area_t = (tx2 - tx1) * (ty2 - ty1)
    area_p = (px2 - px1) * (py2 - py1)
    ov = inter / (area_t + area_p - inter)

    t_iota = jax.lax.broadcasted_iota(jnp.int32, (_O, _P), 0)
    p_iota = jax.lax.broadcasted_iota(jnp.int32, (_O, _P), 1)

    bto = jnp.max(ov, axis=0, keepdims=True)
    bti = jnp.min(jnp.where(ov == bto, t_iota, _O),
                  axis=0, keepdims=True)
    bpo = jnp.max(ov, axis=1, keepdims=True)
    bpi = jnp.min(jnp.where(ov == bpo, p_iota, _P),
                  axis=1, keepdims=True)

    eq = p_iota == bpi
    t_win = jnp.max(jnp.where(eq, t_iota, -1), axis=0, keepdims=True)
    forced = t_win >= 0
    bti = jnp.where(forced, t_win, bti)
    bto = jnp.where(forced, 2.0, bto)

    onehot = (t_iota == bti).astype(jnp.bfloat16)
    dims = (((1,), (0,)), ((), ()))
    matched = (
        jax.lax.dot_general(tgtT_hi, onehot, dims,
                            preferred_element_type=jnp.float32)
        + jax.lax.dot_general(tgtT_lo, onehot, dims,
                              preferred_element_type=jnp.float32))
    mx1 = matched[0:1, :]
    my1 = matched[1:2, :]
    mx2 = matched[2:3, :]
    my2 = matched[3:4, :]
    mlab = matched[4:5, :]

    pos = bto >= _THR
    posf = pos.astype(jnp.float32)
    conf_t = jnp.where(pos, mlab + 1.0, 0.0).astype(jnp.int32)

    g_cx = ((mx1 + mx2) * 0.5 - pcx) / (_V0 * pw)
    g_cy = ((my1 + my2) * 0.5 - pcy) / (_V0 * ph)
    g_w = jnp.log((mx2 - mx1) / pw) / _V1
    g_h = jnp.log((my2 - my1) / ph) / _V1

    loc = loc_ref[0]
    sl1_acc = jnp.zeros((1, _P), jnp.float32)
    for c, g in enumerate((g_cx, g_cy, g_w, g_h)):
        d = loc[c:c + 1, :] - g
        ad = jnp.abs(d)
        sl1_acc = sl1_acc + jnp.where(ad < 1.0, 0.5 * d * d, ad - 0.5)

    cf = conf_ref[0]
    e = jnp.exp(cf)
    c_iota = jax.lax.broadcasted_iota(jnp.int32, (_C, _P), 0)
    cfm = jnp.where(c_iota == conf_t, cf, 0.0)
    s = jnp.sum(e, axis=0, keepdims=True)
    xt = jnp.sum(cfm, axis=0, keepdims=True)
    lse = jnp.log(s)
    ce = lse - xt

    num_pos = jnp.sum(posf)
    ce_pos = jnp.sum(ce * posf)
    ll = jnp.sum(sl1_acc * posf)
    v = jnp.where(pos, 0.0, ce)

    v_ref[...] = v.reshape(1, 1, _P)
    lane = jax.lax.broadcasted_iota(jnp.int32, (1, 128), 1)
    row = jnp.where(lane == 0, ll,
                    jnp.where(lane == 1, ce_pos,
                              jnp.where(lane == 2, num_pos, 0.0)))
    part_ref[...] = row.reshape(1, 1, 128)


def _phase2(v_ref, part_ref, out_ref):
    v = v_ref[...]
    part = part_ref[...]
    num_pos = part[:, 2:3]
    k = jnp.minimum(num_pos * _NEGPOS, jnp.float32(_P - 1))

    bits = jax.lax.bitcast_convert_type(v, jnp.int32)

    def body(_, carry):
        lo, hi = carry
        mid = lo + (hi - lo) // 2
        cnt = jnp.sum((bits > mid).astype(jnp.float32), axis=1, keepdims=True)
        take_hi = cnt < k
        return (jnp.where(take_hi, lo, mid), jnp.where(take_hi, mid, hi))

    lo0 = jnp.full((_B, 1), -1, jnp.int32)
    hi0 = jnp.full((_B, 1), 0x7F800000, jnp.int32)
    _, hi = jax.lax.fori_loop(0, 31, body, (lo0, hi0))

    thr = jax.lax.bitcast_convert_type(hi, jnp.float32)
    gt = v > thr
    cnt_gt = jnp.sum(gt.astype(jnp.float32), axis=1, keepdims=True)
    sum_gt = jnp.sum(jnp.where(gt, v, 0.0), axis=1, keepdims=True)
    topk = sum_gt + (k - cnt_gt) * thr

    ll = jnp.sum(part[:, 0:1])
    ce_sel = jnp.sum(part[:, 1:2]) + jnp.sum(topk)
    n = jnp.sum(num_pos)

    lane = jax.lax.broadcasted_iota(jnp.int32, (8, 128), 1)
    row = jax.lax.broadcasted_iota(jnp.int32, (8, 128), 0)
    out = jnp.where(row == 0,
                    jnp.where(lane == 0, ll,
                              jnp.where(lane == 1, ce_sel,
                                        jnp.where(lane == 2, n, 0.0))),
                    0.0)
    out_ref[...] = out


def kernel(loc_data, conf_data, priors, targets, targets_idx):
    del targets_idx
    pri_t = priors.T
    tgt3 = targets.reshape(_B, _O, 5)
    tgtT = tgt3.transpose(0, 2, 1)
    tgtT_hi = tgtT.astype(jnp.bfloat16)
    tgtT_lo = (tgtT - tgtT_hi.astype(jnp.float32)).astype(jnp.bfloat16)

    v_parts, p_parts = [], []
    for i, _CH in ((0, 8), (8, 24)):
        v_i, part_i = pl.pallas_call(
            _phase1,
            grid=(_CH,),
            in_specs=[
                pl.BlockSpec((1, _O, 5), lambda b: (b, 0, 0)),
                pl.BlockSpec((1, 5, _O), lambda b: (b, 0, 0)),
                pl.BlockSpec((1, 5, _O), lambda b: (b, 0, 0)),
                pl.BlockSpec((4, _P), lambda b: (0, 0)),
                pl.BlockSpec((1, 4, _P), lambda b: (b, 0, 0)),
                pl.BlockSpec((1, _C, _P), lambda b: (b, 0, 0)),
            ],
            out_specs=[
                pl.BlockSpec((1, 1, _P), lambda b: (b, 0, 0)),
                pl.BlockSpec((1, 1, 128), lambda b: (b, 0, 0)),
            ],
            out_shape=[
                jax.ShapeDtypeStruct((_CH, 1, _P), jnp.float32),
                jax.ShapeDtypeStruct((_CH, 1, 128), jnp.float32),
            ],
        )(tgt3[i:i + _CH], tgtT_hi[i:i + _CH], tgtT_lo[i:i + _CH], pri_t,
          loc_data[i:i + _CH].transpose(0, 2, 1),
          conf_data[i:i + _CH].transpose(0, 2, 1))
        v_parts.append(v_i)
        p_parts.append(part_i)
    v = jnp.concatenate(v_parts, axis=0)
    part = jnp.concatenate(p_parts, axis=0)

    out = pl.pallas_call(
        _phase2,
        in_specs=[
            pl.BlockSpec((_B, _P), lambda: (0, 0)),
            pl.BlockSpec((_B, 128), lambda: (0, 0)),
        ],
        out_specs=pl.BlockSpec((8, 128), lambda: (0, 0)),
        out_shape=jax.ShapeDtypeStruct((8, 128), jnp.float32),
    )(v.reshape(_B, _P), part.reshape(_B, 128))

    n = out[0, 2]
    return out[0, 0] / n, out[0, 1] / n


# speedup vs baseline: 1.1248x; 1.1248x over previous
import jax
import jax.numpy as jnp
from jax.experimental import pallas as pl

_C = 21
_B = 32
_P = 8732
_O = 10
_THR = 0.5
_NEGPOS = 3.0
_V0 = 0.1
_V1 = 0.2


def _phase1(tgt_ref, tgtT_hi_ref, tgtT_lo_ref, pri_ref, loc_ref, conf_ref,
            v_ref, part_ref):
    tgt = tgt_ref[0]
    tgtT_hi = tgtT_hi_ref[0]
    tgtT_lo = tgtT_lo_ref[0]
    tx1 = tgt[:, 0:1]
    ty1 = tgt[:, 1:2]
    tx2 = tgt[:, 2:3]
    ty2 = tgt[:, 3:4]

    pri = pri_ref[...]
    pcx = pri[0:1, :]
    pcy = pri[1:2, :]
    pw = pri[2:3, :]
    ph = pri[3:4, :]
    px1 = pcx - pw * 0.5
    py1 = pcy - ph * 0.5
    px2 = pcx + pw * 0.5
    py2 = pcy + ph * 0.5

    iw = jnp.maximum(jnp.minimum(tx2, px2) - jnp.maximum(tx1, px1), 0.0)
    ih = jnp.maximum(jnp.minimum(ty2, py2) - jnp.maximum(ty1, py1), 0.0)
    inter = iw * ih
    area_t = (tx2 - tx1) * (ty2 - ty1)
    area_p = (px2 - px1) * (py2 - py1)
    ov = inter / (area_t + area_p - inter)

    t_iota = jax.lax.broadcasted_iota(jnp.int32, (_O, _P), 0)
    p_iota = jax.lax.broadcasted_iota(jnp.int32, (_O, _P), 1)

    bto = jnp.max(ov, axis=0, keepdims=True)
    bti = jnp.min(jnp.where(ov == bto, t_iota, _O),
                  axis=0, keepdims=True)
    bpo = jnp.max(ov, axis=1, keepdims=True)
    bpi = jnp.min(jnp.where(ov == bpo, p_iota, _P),
                  axis=1, keepdims=True)

    eq = p_iota == bpi
    t_win = jnp.max(jnp.where(eq, t_iota, -1), axis=0, keepdims=True)
    forced = t_win >= 0
    bti = jnp.where(forced, t_win, bti)
    bto = jnp.where(forced, 2.0, bto)

    onehot = (t_iota == bti).astype(jnp.bfloat16)
    dims = (((1,), (0,)), ((), ()))
    matched = (
        jax.lax.dot_general(tgtT_hi, onehot, dims,
                            preferred_element_type=jnp.float32)
        + jax.lax.dot_general(tgtT_lo, onehot, dims,
                              preferred_element_type=jnp.float32))
    mx1 = matched[0:1, :]
    my1 = matched[1:2, :]
    mx2 = matched[2:3, :]
    my2 = matched[3:4, :]
    mlab = matched[4:5, :]

    pos = bto >= _THR
    posf = pos.astype(jnp.float32)
    conf_t = jnp.where(pos, mlab + 1.0, 0.0).astype(jnp.int32)

    g_cx = ((mx1 + mx2) * 0.5 - pcx) / (_V0 * pw)
    g_cy = ((my1 + my2) * 0.5 - pcy) / (_V0 * ph)
    g_w = jnp.log((mx2 - mx1) / pw) / _V1
    g_h = jnp.log((my2 - my1) / ph) / _V1

    loc = loc_ref[0]
    sl1_acc = jnp.zeros((1, _P), jnp.float32)
    for c, g in enumerate((g_cx, g_cy, g_w, g_h)):
        d = loc[c:c + 1, :] - g
        ad = jnp.abs(d)
        sl1_acc = sl1_acc + jnp.where(ad < 1.0, 0.5 * d * d, ad - 0.5)

    cf = conf_ref[0]
    e = jnp.exp(cf)
    c_iota = jax.lax.broadcasted_iota(jnp.int32, (_C, _P), 0)
    cfm = jnp.where(c_iota == conf_t, cf, 0.0)
    s = jnp.sum(e, axis=0, keepdims=True)
    xt = jnp.sum(cfm, axis=0, keepdims=True)
    lse = jnp.log(s)
    ce = lse - xt

    num_pos = jnp.sum(posf)
    ce_pos = jnp.sum(ce * posf)
    ll = jnp.sum(sl1_acc * posf)
    v = jnp.where(pos, 0.0, ce)

    v_ref[...] = v.reshape(1, 1, _P)
    lane = jax.lax.broadcasted_iota(jnp.int32, (1, 128), 1)
    row = jnp.where(lane == 0, ll,
                    jnp.where(lane == 1, ce_pos,
                              jnp.where(lane == 2, num_pos, 0.0)))
    part_ref[...] = row.reshape(1, 1, 128)


def _phase2(v_ref, part_ref, out_ref):
    v = v_ref[...]
    part = part_ref[...]
    num_pos = part[:, 2:3]
    k = jnp.minimum(num_pos * _NEGPOS, jnp.float32(_P - 1))

    bits = jax.lax.bitcast_convert_type(v, jnp.int32)

    def body(_, carry):
        lo, hi = carry
        mid = lo + (hi - lo) // 2
        cnt = jnp.sum((bits > mid).astype(jnp.float32), axis=1, keepdims=True)
        take_hi = cnt < k
        return (jnp.where(take_hi, lo, mid), jnp.where(take_hi, mid, hi))

    lo0 = jnp.full((_B, 1), -1, jnp.int32)
    hi0 = jnp.full((_B, 1), 0x7F800000, jnp.int32)
    _, hi = jax.lax.fori_loop(0, 31, body, (lo0, hi0))

    thr = jax.lax.bitcast_convert_type(hi, jnp.float32)
    gt = v > thr
    cnt_gt = jnp.sum(gt.astype(jnp.float32), axis=1, keepdims=True)
    sum_gt = jnp.sum(jnp.where(gt, v, 0.0), axis=1, keepdims=True)
    topk = sum_gt + (k - cnt_gt) * thr

    ll = jnp.sum(part[:, 0:1])
    ce_sel = jnp.sum(part[:, 1:2]) + jnp.sum(topk)
    n = jnp.sum(num_pos)

    lane = jax.lax.broadcasted_iota(jnp.int32, (8, 128), 1)
    row = jax.lax.broadcasted_iota(jnp.int32, (8, 128), 0)
    out = jnp.where(row == 0,
                    jnp.where(lane == 0, ll,
                              jnp.where(lane == 1, ce_sel,
                                        jnp.where(lane == 2, n, 0.0))),
                    0.0)
    out_ref[...] = out


def kernel(loc_data, conf_data, priors, targets, targets_idx):
    del targets_idx
    pri_t = priors.T
    tgt3 = targets.reshape(_B, _O, 5)
    tgtT = tgt3.transpose(0, 2, 1)
    tgtT_hi = tgtT.astype(jnp.bfloat16)
    tgtT_lo = (tgtT - tgtT_hi.astype(jnp.float32)).astype(jnp.bfloat16)

    _CH = 16
    v_parts, p_parts = [], []
    for i in range(0, _B, _CH):
        v_i, part_i = pl.pallas_call(
            _phase1,
            grid=(_CH,),
            in_specs=[
                pl.BlockSpec((1, _O, 5), lambda b: (b, 0, 0)),
                pl.BlockSpec((1, 5, _O), lambda b: (b, 0, 0)),
                pl.BlockSpec((1, 5, _O), lambda b: (b, 0, 0)),
                pl.BlockSpec((4, _P), lambda b: (0, 0)),
                pl.BlockSpec((1, 4, _P), lambda b: (b, 0, 0)),
                pl.BlockSpec((1, _C, _P), lambda b: (b, 0, 0)),
            ],
            out_specs=[
                pl.BlockSpec((1, 1, _P), lambda b: (b, 0, 0)),
                pl.BlockSpec((1, 1, 128), lambda b: (b, 0, 0)),
            ],
            out_shape=[
                jax.ShapeDtypeStruct((_CH, 1, _P), jnp.float32),
                jax.ShapeDtypeStruct((_CH, 1, 128), jnp.float32),
            ],
        )(tgt3[i:i + _CH], tgtT_hi[i:i + _CH], tgtT_lo[i:i + _CH], pri_t,
          loc_data[i:i + _CH].transpose(0, 2, 1),
          conf_data[i:i + _CH].transpose(0, 2, 1))
        v_parts.append(v_i)
        p_parts.append(part_i)
    v = jnp.concatenate(v_parts, axis=0)
    part = jnp.concatenate(p_parts, axis=0)

    out = pl.pallas_call(
        _phase2,
        in_specs=[
            pl.BlockSpec((_B, _P), lambda: (0, 0)),
            pl.BlockSpec((_B, 128), lambda: (0, 0)),
        ],
        out_specs=pl.BlockSpec((8, 128), lambda: (0, 0)),
        out_shape=jax.ShapeDtypeStruct((8, 128), jnp.float32),
    )(v.reshape(_B, _P), part.reshape(_B, 128))

    n = out[0, 2]
    return out[0, 0] / n, out[0, 1] / n


# speedup vs baseline: 1.1541x; 1.0261x over previous
import jax
import jax.numpy as jnp
from jax.experimental import pallas as pl

_C = 21
_B = 32
_P = 8732
_O = 10
_THR = 0.5
_NEGPOS = 3.0
_V0 = 0.1
_V1 = 0.2


def _phase1(tgt_ref, tgtT_hi_ref, tgtT_lo_ref, pri_ref, loc_ref, conf_ref,
            v_ref, part_ref):
    tgt = tgt_ref[0]
    tgtT_hi = tgtT_hi_ref[0]
    tgtT_lo = tgtT_lo_ref[0]
    tx1 = tgt[:, 0:1]
    ty1 = tgt[:, 1:2]
    tx2 = tgt[:, 2:3]
    ty2 = tgt[:, 3:4]

    pri = pri_ref[...]
    pcx = pri[0:1, :]
    pcy = pri[1:2, :]
    pw = pri[2:3, :]
    ph = pri[3:4, :]
    px1 = pcx - pw * 0.5
    py1 = pcy - ph * 0.5
    px2 = pcx + pw * 0.5
    py2 = pcy + ph * 0.5

    iw = jnp.maximum(jnp.minimum(tx2, px2) - jnp.maximum(tx1, px1), 0.0)
    ih = jnp.maximum(jnp.minimum(ty2, py2) - jnp.maximum(ty1, py1), 0.0)
    inter = iw * ih
    area_t = (tx2 - tx1) * (ty2 - ty1)
    area_p = (px2 - px1) * (py2 - py1)
    ov = inter / (area_t + area_p - inter)

    t_iota = jax.lax.broadcasted_iota(jnp.int32, (_O, _P), 0)
    p_iota = jax.lax.broadcasted_iota(jnp.int32, (_O, _P), 1)

    bto = jnp.max(ov, axis=0, keepdims=True)
    bti = jnp.min(jnp.where(ov == bto, t_iota, _O),
                  axis=0, keepdims=True)
    bpo = jnp.max(ov, axis=1, keepdims=True)
    bpi = jnp.min(jnp.where(ov == bpo, p_iota, _P),
                  axis=1, keepdims=True)

    eq = p_iota == bpi
    t_win = jnp.max(jnp.where(eq, t_iota, -1), axis=0, keepdims=True)
    forced = t_win >= 0
    bti = jnp.where(forced, t_win, bti)
    bto = jnp.where(forced, 2.0, bto)

    onehot = (t_iota == bti).astype(jnp.bfloat16)
    dims = (((1,), (0,)), ((), ()))
    matched = (
        jax.lax.dot_general(tgtT_hi, onehot, dims,
                            preferred_element_type=jnp.float32)
        + jax.lax.dot_general(tgtT_lo, onehot, dims,
                              preferred_element_type=jnp.float32))
    mx1 = matched[0:1, :]
    my1 = matched[1:2, :]
    mx2 = matched[2:3, :]
    my2 = matched[3:4, :]
    mlab = matched[4:5, :]

    pos = bto >= _THR
    posf = pos.astype(jnp.float32)
    conf_t = jnp.where(pos, mlab + 1.0, 0.0).astype(jnp.int32)

    g_cx = ((mx1 + mx2) * 0.5 - pcx) / (_V0 * pw)
    g_cy = ((my1 + my2) * 0.5 - pcy) / (_V0 * ph)
    g_w = jnp.log((mx2 - mx1) / pw) / _V1
    g_h = jnp.log((my2 - my1) / ph) / _V1

    loc = loc_ref[0]
    sl1_acc = jnp.zeros((1, _P), jnp.float32)
    for c, g in enumerate((g_cx, g_cy, g_w, g_h)):
        d = loc[c:c + 1, :] - g
        ad = jnp.abs(d)
        sl1_acc = sl1_acc + jnp.where(ad < 1.0, 0.5 * d * d, ad - 0.5)

    cf = conf_ref[0]
    e = jnp.exp(cf)
    c_iota = jax.lax.broadcasted_iota(jnp.int32, (_C, _P), 0)
    cfm = jnp.where(c_iota == conf_t, cf, 0.0)
    s = jnp.sum(e, axis=0, keepdims=True)
    xt = jnp.sum(cfm, axis=0, keepdims=True)
    lse = jnp.log(s)
    ce = lse - xt

    stk = jnp.concatenate([sl1_acc, ce, jnp.ones((1, _P), jnp.float32)],
                          axis=0) * posf
    sums = jnp.sum(stk, axis=1, keepdims=True)
    v = jnp.where(pos, 0.0, ce)

    v_ref[...] = v.reshape(1, 1, _P)
    lane = jax.lax.broadcasted_iota(jnp.int32, (1, 128), 1)
    row = jnp.where(lane == 0, sums[0:1, 0:1],
                    jnp.where(lane == 1, sums[1:2, 0:1],
                              jnp.where(lane == 2, sums[2:3, 0:1], 0.0)))
    part_ref[...] = row.reshape(1, 1, 128)


def _phase2(v_ref, part_ref, out_ref):
    v = v_ref[...]
    part = part_ref[...]
    num_pos = part[:, 2:3]
    k = jnp.minimum(num_pos * _NEGPOS, jnp.float32(_P - 1))

    bits = jax.lax.bitcast_convert_type(v, jnp.int32)

    def body(_, carry):
        lo, hi = carry
        mid = lo + (hi - lo) // 2
        cnt = jnp.sum((bits > mid).astype(jnp.float32), axis=1, keepdims=True)
        take_hi = cnt < k
        return (jnp.where(take_hi, lo, mid), jnp.where(take_hi, mid, hi))

    lo0 = jnp.full((_B, 1), -1, jnp.int32)
    hi0 = jnp.full((_B, 1), 0x7F800000, jnp.int32)
    _, hi = jax.lax.fori_loop(0, 31, body, (lo0, hi0))

    thr = jax.lax.bitcast_convert_type(hi, jnp.float32)
    gt = v > thr
    cnt_gt = jnp.sum(gt.astype(jnp.float32), axis=1, keepdims=True)
    sum_gt = jnp.sum(jnp.where(gt, v, 0.0), axis=1, keepdims=True)
    topk = sum_gt + (k - cnt_gt) * thr

    ll = jnp.sum(part[:, 0:1])
    ce_sel = jnp.sum(part[:, 1:2]) + jnp.sum(topk)
    n = jnp.sum(num_pos)

    lane = jax.lax.broadcasted_iota(jnp.int32, (8, 128), 1)
    row = jax.lax.broadcasted_iota(jnp.int32, (8, 128), 0)
    out = jnp.where(row == 0,
                    jnp.where(lane == 0, ll,
                              jnp.where(lane == 1, ce_sel,
                                        jnp.where(lane == 2, n, 0.0))),
                    0.0)
    out_ref[...] = out


def kernel(loc_data, conf_data, priors, targets, targets_idx):
    del targets_idx
    pri_t = priors.T
    tgt3 = targets.reshape(_B, _O, 5)
    tgtT = tgt3.transpose(0, 2, 1)
    tgtT_hi = tgtT.astype(jnp.bfloat16)
    tgtT_lo = (tgtT - tgtT_hi.astype(jnp.float32)).astype(jnp.bfloat16)

    _CH = 16
    v_parts, p_parts = [], []
    for i in range(0, _B, _CH):
        v_i, part_i = pl.pallas_call(
            _phase1,
            grid=(_CH,),
            in_specs=[
                pl.BlockSpec((1, _O, 5), lambda b: (b, 0, 0)),
                pl.BlockSpec((1, 5, _O), lambda b: (b, 0, 0)),
                pl.BlockSpec((1, 5, _O), lambda b: (b, 0, 0)),
                pl.BlockSpec((4, _P), lambda b: (0, 0)),
                pl.BlockSpec((1, 4, _P), lambda b: (b, 0, 0)),
                pl.BlockSpec((1, _C, _P), lambda b: (b, 0, 0)),
            ],
            out_specs=[
                pl.BlockSpec((1, 1, _P), lambda b: (b, 0, 0)),
                pl.BlockSpec((1, 1, 128), lambda b: (b, 0, 0)),
            ],
            out_shape=[
                jax.ShapeDtypeStruct((_CH, 1, _P), jnp.float32),
                jax.ShapeDtypeStruct((_CH, 1, 128), jnp.float32),
            ],
        )(tgt3[i:i + _CH], tgtT_hi[i:i + _CH], tgtT_lo[i:i + _CH], pri_t,
          loc_data[i:i + _CH].transpose(0, 2, 1),
          conf_data[i:i + _CH].transpose(0, 2, 1))
        v_parts.append(v_i)
        p_parts.append(part_i)
    v = jnp.concatenate(v_parts, axis=0)
    part = jnp.concatenate(p_parts, axis=0)

    out = pl.pallas_call(
        _phase2,
        in_specs=[
            pl.BlockSpec((_B, _P), lambda: (0, 0)),
            pl.BlockSpec((_B, 128), lambda: (0, 0)),
        ],
        out_specs=pl.BlockSpec((8, 128), lambda: (0, 0)),
        out_shape=jax.ShapeDtypeStruct((8, 128), jnp.float32),
    )(v.reshape(_B, _P), part.reshape(_B, 128))

    n = out[0, 2]
    return out[0, 0] / n, out[0, 1] / n
